# baseline (speedup 1.0000x reference)
import functools
import math

import jax
import jax.numpy as jnp
from jax import lax
from jax.experimental import pallas as pl
from jax.experimental.pallas import tpu as pltpu
from jax.experimental.pallas import tpu_sc as plsc

_N = 10000
_E = 320000
_D = 128
_HEAD_DIM = 32

_NC = 2
_NS = 16
_L = 16
_NW = _NC * _NS
_C = 80
_NCHT = _E // _C
_CPW = _NCHT // _NW
_NBLK = _N // 8
_BPS = _NBLK // _NS

_INV_SQRT_HD = 1.0 / math.sqrt(float(_HEAD_DIM))

_GATHER_DNUMS = lax.GatherDimensionNumbers(
    offset_dims=(), collapsed_slice_dims=(0,), start_index_map=(0,))


def _lane_shuffle(x, idx2d):
    return lax.gather(x, idx2d, _GATHER_DNUMS, slice_sizes=(1,),
                      mode=lax.GatherScatterMode.PROMISE_IN_BOUNDS)


def _pack_cols(arr):
    bits = lax.bitcast_convert_type(arr.astype(jnp.bfloat16), jnp.uint16)
    b32 = bits.astype(jnp.int32)
    return b32[:, : _D // 2] | (b32[:, _D // 2:] << 16)


def _qkv_body(x_ref, wq_ref, wk_ref, wv_ref, bq_ref, bk_ref, bv_ref,
              q_ref, kv_ref):
    xb = x_ref[...]
    q_ref[...] = jnp.dot(xb, wq_ref[...], preferred_element_type=jnp.float32,
                         precision=lax.Precision.HIGHEST) + bq_ref[...]
    k = jnp.dot(xb, wk_ref[...], preferred_element_type=jnp.float32,
                precision=lax.Precision.HIGHEST) + bk_ref[...]
    v = jnp.dot(xb, wv_ref[...], preferred_element_type=jnp.float32,
                precision=lax.Precision.HIGHEST) + bv_ref[...]
    kv = jnp.concatenate([_pack_cols(k), _pack_cols(v)], axis=1)
    kv_ref[...] = lax.bitcast_convert_type(kv, jnp.float32)


def _qkv_proj(x2d, Wq, bq, Wk, bk, Wv, bv):
    blk = 1000
    grid = (_N // blk,)
    row_spec = pl.BlockSpec((blk, _D), lambda i: (i, 0))
    w_spec = pl.BlockSpec((_D, _D), lambda i: (0, 0))
    b_spec = pl.BlockSpec((1, _D), lambda i: (0, 0))
    out = jax.ShapeDtypeStruct((_N, _D), jnp.float32)
    return pl.pallas_call(
        _qkv_body,
        grid=grid,
        in_specs=[row_spec, w_spec, w_spec, w_spec, b_spec, b_spec, b_spec],
        out_specs=[row_spec, row_spec],
        out_shape=[out, out],
    )(x2d, Wq, Wk, Wv, bq.reshape(1, _D), bk.reshape(1, _D), bv.reshape(1, _D))


def _out_body(p_ref, wo_ref, bo_ref, o_ref):
    s = p_ref[0] + p_ref[1]
    o_ref[...] = jnp.dot(s, wo_ref[...], preferred_element_type=jnp.float32,
                         precision=lax.Precision.HIGHEST) + bo_ref[...]


def _out_proj(partials, Wo, bo):
    blk = 1000
    grid = (_N // blk,)
    return pl.pallas_call(
        _out_body,
        grid=grid,
        in_specs=[
            pl.BlockSpec((2, blk, _D), lambda i: (0, i, 0)),
            pl.BlockSpec((_D, _D), lambda i: (0, 0)),
            pl.BlockSpec((1, _D), lambda i: (0, 0)),
        ],
        out_specs=pl.BlockSpec((blk, _D), lambda i: (i, 0)),
        out_shape=jax.ShapeDtypeStruct((_N, _D), jnp.float32),
    )(partials, Wo, bo.reshape(1, _D))


def _sc_edges(q_hbm, kv_hbm, ei_hbm, out_hbm,
              ib0, ib1, ib2, ib3, qd0, kv0, qd1, kv1,
              zero_v, acc_sh, gsem0, gsem1, ssem0, ssem1,
              isem0, isem1, isem2, isem3):
    c = lax.axis_index("c")
    s = lax.axis_index("s")
    wid = c * _NS + s
    zvec = jnp.zeros((_L,), jnp.float32)

    nb = jnp.where(s < 2, _BPS + 1, _BPS)
    b0 = s * _BPS + jnp.minimum(s, 2)

    for i in range(8):
        for j in range(_D // _L):
            zero_v[i, pl.ds(j * _L, _L)] = zvec

    def zblk(i, _):
        r0 = pl.multiple_of((b0 + i) * 8, 8)
        pltpu.async_copy(zero_v, acc_sh.at[pl.ds(r0, 8)], isem0)
        return 0

    def zdrain(i, _):
        r0 = pl.multiple_of((b0 + i) * 8, 8)
        pltpu.make_async_copy(zero_v, acc_sh.at[pl.ds(r0, 8)], isem0).wait()
        return 0

    lax.fori_loop(0, nb, zblk, 0)
    lax.fori_loop(0, nb, zdrain, 0)
    plsc.subcore_barrier()

    lanes = lax.iota(jnp.int32, _L)
    bfly = [jnp.bitwise_xor(lanes, jnp.int32(w))[:, None] for w in (8, 4, 2, 1)]

    def idx_start(i, ib, isem):
        g = wid * _CPW + jnp.minimum(i, _CPW - 1)
        pltpu.async_copy(ei_hbm.at[g], ib, isem)

    def idx_wait(i, ib, isem):
        g = wid * _CPW + jnp.minimum(i, _CPW - 1)
        pltpu.make_async_copy(ei_hbm.at[g], ib, isem).wait()

    def issue_gathers(ib, qd, kv, gsem):
        pltpu.async_copy(q_hbm.at[ib.at[1]], qd, gsem)
        pltpu.async_copy(kv_hbm.at[ib.at[0]], kv, gsem)

    def wait_gathers(ib, qd, kv, gsem):
        pltpu.make_async_copy(q_hbm.at[ib.at[1]], qd, gsem).wait()
        pltpu.make_async_copy(kv_hbm.at[ib.at[0]], kv, gsem).wait()

    def compute(qd, kv):
        @plsc.parallel_loop(0, _C, unroll=4)
        def edge(e):
            acc0 = jnp.zeros((_L,), jnp.float32)
            acc1 = jnp.zeros((_L,), jnp.float32)
            hi_mask = jnp.full((_L,), -65536, jnp.int32)
            half = _D // 2
            for j in range(_D // (2 * _L)):
                kw = lax.bitcast_convert_type(kv[e, pl.ds(j * _L, _L)], jnp.int32)
                qa = qd[e, pl.ds(j * _L, _L)]
                qb = qd[e, pl.ds(half + j * _L, _L)]
                ka = lax.bitcast_convert_type(jnp.left_shift(kw, 16), jnp.float32)
                kb = lax.bitcast_convert_type(jnp.bitwise_and(kw, hi_mask), jnp.float32)
                acc0 = acc0 + qa * ka
                acc1 = acc1 + qb * kb
            acc = acc0 + acc1
            for idx in bfly:
                acc = acc + _lane_shuffle(acc, idx)
            a = 1.0 / (1.0 + jnp.exp(acc * (-_INV_SQRT_HD)))
            for j in range(_D // (2 * _L)):
                vw = lax.bitcast_convert_type(
                    kv[e, pl.ds(half + j * _L, _L)], jnp.int32)
                va = lax.bitcast_convert_type(jnp.left_shift(vw, 16), jnp.float32)
                vb = lax.bitcast_convert_type(jnp.bitwise_and(vw, hi_mask), jnp.float32)
                kv[e, pl.ds(j * _L, _L)] = va * a
                kv[e, pl.ds(half + j * _L, _L)] = vb * a

    def scatter(ib, ms, ssem):
        pltpu.async_copy(ms, acc_sh.at[ib.at[1]], ssem, add=True)

    def wait_scatter(ib, ms, ssem):
        pltpu.make_async_copy(ms, acc_sh.at[ib.at[1]], ssem).wait()

    def slot(m, ibm, isemm, ibn, isemn, qd, kvb, gsem, ssem):
        wait_gathers(ibm, qd, kvb, gsem)
        compute(qd, kvb)
        scatter(ibm, kvb, ssem)
        wait_scatter(ibm, kvb, ssem)
        idx_wait(m + 2, ibn, isemn)
        issue_gathers(ibn, qd, kvb, gsem)
        idx_start(m + 4, ibm, isemm)

    idx_start(0, ib0, isem0)
    idx_start(1, ib1, isem1)
    idx_start(2, ib2, isem2)
    idx_start(3, ib3, isem3)
    idx_wait(0, ib0, isem0)
    issue_gathers(ib0, qd0, kv0, gsem0)
    idx_wait(1, ib1, isem1)
    issue_gathers(ib1, qd1, kv1, gsem1)

    def quad(j, _):
        m = 4 * j
        slot(m + 0, ib0, isem0, ib2, isem2, qd0, kv0, gsem0, ssem0)
        slot(m + 1, ib1, isem1, ib3, isem3, qd1, kv1, gsem1, ssem1)
        slot(m + 2, ib2, isem2, ib0, isem0, qd0, kv0, gsem0, ssem0)
        slot(m + 3, ib3, isem3, ib1, isem1, qd1, kv1, gsem1, ssem1)
        return 0

    lax.fori_loop(0, _CPW // 4, quad, 0)

    wait_gathers(ib0, qd0, kv0, gsem0)
    compute(qd0, kv0)
    scatter(ib0, kv0, ssem0)
    wait_scatter(ib0, kv0, ssem0)
    wait_gathers(ib1, qd1, kv1, gsem1)
    idx_wait(_CPW + 1, ib2, isem2)
    idx_wait(_CPW + 2, ib3, isem3)

    plsc.subcore_barrier()

    r0 = pl.multiple_of(b0 * 8, 8)

    @pl.when(s < 2)
    def _():
        pltpu.sync_copy(acc_sh.at[pl.ds(r0, (_BPS + 1) * 8)],
                        out_hbm.at[c, pl.ds(r0, (_BPS + 1) * 8)])

    @pl.when(s >= 2)
    def _():
        pltpu.sync_copy(acc_sh.at[pl.ds(r0, _BPS * 8)],
                        out_hbm.at[c, pl.ds(r0, _BPS * 8)])


def _sc_gat(q, kv, ei):
    mesh = plsc.VectorSubcoreMesh(core_axis_name="c", subcore_axis_name="s")
    f = pl.kernel(
        _sc_edges,
        mesh=mesh,
        out_type=jax.ShapeDtypeStruct((_NC, _N, _D), jnp.float32),
        scratch_types=[
            pltpu.VMEM((2, _C), jnp.int32),
            pltpu.VMEM((2, _C), jnp.int32),
            pltpu.VMEM((2, _C), jnp.int32),
            pltpu.VMEM((2, _C), jnp.int32),
            pltpu.VMEM((_C, _D), jnp.float32),
            pltpu.VMEM((_C, _D), jnp.float32),
            pltpu.VMEM((_C, _D), jnp.float32),
            pltpu.VMEM((_C, _D), jnp.float32),
            pltpu.VMEM((8, _D), jnp.float32),
            pltpu.VMEM_SHARED((_N, _D), jnp.float32),
            pltpu.SemaphoreType.DMA,
            pltpu.SemaphoreType.DMA,
            pltpu.SemaphoreType.DMA,
            pltpu.SemaphoreType.DMA,
            pltpu.SemaphoreType.DMA,
            pltpu.SemaphoreType.DMA,
            pltpu.SemaphoreType.DMA,
            pltpu.SemaphoreType.DMA,
        ],
    )
    return f(q, kv, ei)


def kernel(x, edge_index, Wq, bq, Wk, bk, Wv, bv, Wo, bo):
    x2d = x.reshape(_N, _D)
    q, kv = _qkv_proj(x2d, Wq, bq, Wk, bk, Wv, bv)
    ei = edge_index.reshape(2, _NCHT, _C).transpose(1, 0, 2)
    partials = _sc_gat(q, kv, ei)
    out = _out_proj(partials, Wo, bo)
    return out.reshape(1, _N, _D)


# speedup vs baseline: 1.0782x; 1.0782x over previous
import functools
import math

import jax
import jax.numpy as jnp
from jax import lax
from jax.experimental import pallas as pl
from jax.experimental.pallas import tpu as pltpu
from jax.experimental.pallas import tpu_sc as plsc

_N = 10000
_E = 320000
_D = 128
_HEAD_DIM = 32

_NC = 2
_NS = 16
_L = 16
_NW = _NC * _NS
_C = 80
_NCHT = _E // _C
_CPW = _NCHT // _NW
_NBLK = _N // 8
_BPS = _NBLK // _NS

_INV_SQRT_HD = 1.0 / math.sqrt(float(_HEAD_DIM))

_GATHER_DNUMS = lax.GatherDimensionNumbers(
    offset_dims=(), collapsed_slice_dims=(0,), start_index_map=(0,))


def _lane_shuffle(x, idx2d):
    return lax.gather(x, idx2d, _GATHER_DNUMS, slice_sizes=(1,),
                      mode=lax.GatherScatterMode.PROMISE_IN_BOUNDS)


def _pack_cols(arr):
    bits = lax.bitcast_convert_type(arr.astype(jnp.bfloat16), jnp.uint16)
    b32 = bits.astype(jnp.int32)
    return b32[:, : _D // 2] | (b32[:, _D // 2:] << 16)


def _qkv_body(x_ref, wq_ref, wk_ref, wv_ref, bq_ref, bk_ref, bv_ref,
              q_ref, kv_ref):
    xb = x_ref[...]
    q_ref[...] = jnp.dot(xb, wq_ref[...], preferred_element_type=jnp.float32,
                         precision=lax.Precision.HIGHEST) + bq_ref[...]
    k = jnp.dot(xb, wk_ref[...], preferred_element_type=jnp.float32,
                precision=lax.Precision.HIGHEST) + bk_ref[...]
    v = jnp.dot(xb, wv_ref[...], preferred_element_type=jnp.float32,
                precision=lax.Precision.HIGHEST) + bv_ref[...]
    kv = jnp.concatenate([_pack_cols(k), _pack_cols(v)], axis=1)
    kv_ref[...] = lax.bitcast_convert_type(kv, jnp.float32)


def _qkv_proj(x2d, Wq, bq, Wk, bk, Wv, bv):
    blk = 2000
    grid = (_N // blk,)
    row_spec = pl.BlockSpec((blk, _D), lambda i: (i, 0))
    w_spec = pl.BlockSpec((_D, _D), lambda i: (0, 0))
    b_spec = pl.BlockSpec((1, _D), lambda i: (0, 0))
    out = jax.ShapeDtypeStruct((_N, _D), jnp.float32)
    return pl.pallas_call(
        _qkv_body,
        grid=grid,
        in_specs=[row_spec, w_spec, w_spec, w_spec, b_spec, b_spec, b_spec],
        out_specs=[row_spec, row_spec],
        out_shape=[out, out],
    )(x2d, Wq, Wk, Wv, bq.reshape(1, _D), bk.reshape(1, _D), bv.reshape(1, _D))


def _out_body(p_ref, wo_ref, bo_ref, o_ref):
    s = p_ref[0] + p_ref[1]
    o_ref[...] = jnp.dot(s, wo_ref[...], preferred_element_type=jnp.float32,
                         precision=lax.Precision.HIGHEST) + bo_ref[...]


def _out_proj(partials, Wo, bo):
    blk = 2000
    grid = (_N // blk,)
    return pl.pallas_call(
        _out_body,
        grid=grid,
        in_specs=[
            pl.BlockSpec((2, blk, _D), lambda i: (0, i, 0)),
            pl.BlockSpec((_D, _D), lambda i: (0, 0)),
            pl.BlockSpec((1, _D), lambda i: (0, 0)),
        ],
        out_specs=pl.BlockSpec((blk, _D), lambda i: (i, 0)),
        out_shape=jax.ShapeDtypeStruct((_N, _D), jnp.float32),
    )(partials, Wo, bo.reshape(1, _D))


def _sc_edges(q_hbm, kv_hbm, ei_hbm, out_hbm,
              ib0, ib1, ib2, ib3, qd0, kv0, qd1, kv1,
              zero_v, acc_sh, gsem0, gsem1, ssem0, ssem1,
              isem0, isem1, isem2, isem3):
    c = lax.axis_index("c")
    s = lax.axis_index("s")
    wid = c * _NS + s
    zvec = jnp.zeros((_L,), jnp.float32)

    nb = jnp.where(s < 2, _BPS + 1, _BPS)
    b0 = s * _BPS + jnp.minimum(s, 2)

    for i in range(8):
        for j in range(_D // _L):
            zero_v[i, pl.ds(j * _L, _L)] = zvec

    def zblk(i, _):
        r0 = pl.multiple_of((b0 + i) * 8, 8)
        pltpu.async_copy(zero_v, acc_sh.at[pl.ds(r0, 8)], isem0)
        return 0

    def zdrain(i, _):
        r0 = pl.multiple_of((b0 + i) * 8, 8)
        pltpu.make_async_copy(zero_v, acc_sh.at[pl.ds(r0, 8)], isem0).wait()
        return 0

    lax.fori_loop(0, nb, zblk, 0)
    lax.fori_loop(0, nb, zdrain, 0)
    plsc.subcore_barrier()

    lanes = lax.iota(jnp.int32, _L)
    bfly = [jnp.bitwise_xor(lanes, jnp.int32(w))[:, None] for w in (8, 4, 2, 1)]

    def idx_start(i, ib, isem):
        g = wid * _CPW + jnp.minimum(i, _CPW - 1)
        pltpu.async_copy(ei_hbm.at[g], ib, isem)

    def idx_wait(i, ib, isem):
        g = wid * _CPW + jnp.minimum(i, _CPW - 1)
        pltpu.make_async_copy(ei_hbm.at[g], ib, isem).wait()

    def issue_gathers(ib, qd, kv, gsem):
        pltpu.async_copy(q_hbm.at[ib.at[1]], qd, gsem)
        pltpu.async_copy(kv_hbm.at[ib.at[0]], kv, gsem)

    def wait_gathers(ib, qd, kv, gsem):
        pltpu.make_async_copy(q_hbm.at[ib.at[1]], qd, gsem).wait()
        pltpu.make_async_copy(kv_hbm.at[ib.at[0]], kv, gsem).wait()

    def compute(qd, kv):
        @plsc.parallel_loop(0, _C, unroll=2)
        def edge(e):
            acc0 = jnp.zeros((_L,), jnp.float32)
            acc1 = jnp.zeros((_L,), jnp.float32)
            hi_mask = jnp.full((_L,), -65536, jnp.int32)
            half = _D // 2
            for j in range(_D // (2 * _L)):
                kw = lax.bitcast_convert_type(kv[e, pl.ds(j * _L, _L)], jnp.int32)
                qa = qd[e, pl.ds(j * _L, _L)]
                qb = qd[e, pl.ds(half + j * _L, _L)]
                ka = lax.bitcast_convert_type(jnp.left_shift(kw, 16), jnp.float32)
                kb = lax.bitcast_convert_type(jnp.bitwise_and(kw, hi_mask), jnp.float32)
                acc0 = acc0 + qa * ka
                acc1 = acc1 + qb * kb
            acc = acc0 + acc1
            for idx in bfly:
                acc = acc + _lane_shuffle(acc, idx)
            a = 1.0 / (1.0 + jnp.exp(acc * (-_INV_SQRT_HD)))
            for j in range(_D // (2 * _L)):
                vw = lax.bitcast_convert_type(
                    kv[e, pl.ds(half + j * _L, _L)], jnp.int32)
                va = lax.bitcast_convert_type(jnp.left_shift(vw, 16), jnp.float32)
                vb = lax.bitcast_convert_type(jnp.bitwise_and(vw, hi_mask), jnp.float32)
                kv[e, pl.ds(j * _L, _L)] = va * a
                kv[e, pl.ds(half + j * _L, _L)] = vb * a

    def scatter(ib, ms, ssem):
        pltpu.async_copy(ms, acc_sh.at[ib.at[1]], ssem, add=True)

    def wait_scatter(ib, ms, ssem):
        pltpu.make_async_copy(ms, acc_sh.at[ib.at[1]], ssem).wait()

    def slot(m, ibm, isemm, ibn, isemn, qd, kvb, gsem, ssem):
        wait_gathers(ibm, qd, kvb, gsem)
        compute(qd, kvb)
        scatter(ibm, kvb, ssem)
        wait_scatter(ibm, kvb, ssem)
        idx_wait(m + 2, ibn, isemn)
        issue_gathers(ibn, qd, kvb, gsem)
        idx_start(m + 4, ibm, isemm)

    idx_start(0, ib0, isem0)
    idx_start(1, ib1, isem1)
    idx_start(2, ib2, isem2)
    idx_start(3, ib3, isem3)
    idx_wait(0, ib0, isem0)
    issue_gathers(ib0, qd0, kv0, gsem0)
    idx_wait(1, ib1, isem1)
    issue_gathers(ib1, qd1, kv1, gsem1)

    def quad(j, _):
        m = 4 * j
        slot(m + 0, ib0, isem0, ib2, isem2, qd0, kv0, gsem0, ssem0)
        slot(m + 1, ib1, isem1, ib3, isem3, qd1, kv1, gsem1, ssem1)
        slot(m + 2, ib2, isem2, ib0, isem0, qd0, kv0, gsem0, ssem0)
        slot(m + 3, ib3, isem3, ib1, isem1, qd1, kv1, gsem1, ssem1)
        return 0

    lax.fori_loop(0, _CPW // 4, quad, 0)

    wait_gathers(ib0, qd0, kv0, gsem0)
    compute(qd0, kv0)
    scatter(ib0, kv0, ssem0)
    wait_scatter(ib0, kv0, ssem0)
    wait_gathers(ib1, qd1, kv1, gsem1)
    idx_wait(_CPW + 1, ib2, isem2)
    idx_wait(_CPW + 2, ib3, isem3)

    plsc.subcore_barrier()

    r0 = pl.multiple_of(b0 * 8, 8)

    @pl.when(s < 2)
    def _():
        pltpu.sync_copy(acc_sh.at[pl.ds(r0, (_BPS + 1) * 8)],
                        out_hbm.at[c, pl.ds(r0, (_BPS + 1) * 8)])

    @pl.when(s >= 2)
    def _():
        pltpu.sync_copy(acc_sh.at[pl.ds(r0, _BPS * 8)],
                        out_hbm.at[c, pl.ds(r0, _BPS * 8)])


def _sc_gat(q, kv, ei):
    mesh = plsc.VectorSubcoreMesh(core_axis_name="c", subcore_axis_name="s")
    f = pl.kernel(
        _sc_edges,
        mesh=mesh,
        out_type=jax.ShapeDtypeStruct((_NC, _N, _D), jnp.float32),
        scratch_types=[
            pltpu.VMEM((2, _C), jnp.int32),
            pltpu.VMEM((2, _C), jnp.int32),
            pltpu.VMEM((2, _C), jnp.int32),
            pltpu.VMEM((2, _C), jnp.int32),
            pltpu.VMEM((_C, _D), jnp.float32),
            pltpu.VMEM((_C, _D), jnp.float32),
            pltpu.VMEM((_C, _D), jnp.float32),
            pltpu.VMEM((_C, _D), jnp.float32),
            pltpu.VMEM((8, _D), jnp.float32),
            pltpu.VMEM_SHARED((_N, _D), jnp.float32),
            pltpu.SemaphoreType.DMA,
            pltpu.SemaphoreType.DMA,
            pltpu.SemaphoreType.DMA,
            pltpu.SemaphoreType.DMA,
            pltpu.SemaphoreType.DMA,
            pltpu.SemaphoreType.DMA,
            pltpu.SemaphoreType.DMA,
            pltpu.SemaphoreType.DMA,
        ],
    )
    return f(q, kv, ei)


def kernel(x, edge_index, Wq, bq, Wk, bk, Wv, bv, Wo, bo):
    x2d = x.reshape(_N, _D)
    q, kv = _qkv_proj(x2d, Wq, bq, Wk, bk, Wv, bv)
    ei = edge_index.reshape(2, _NCHT, _C).transpose(1, 0, 2)
    partials = _sc_gat(q, kv, ei)
    out = _out_proj(partials, Wo, bo)
    return out.reshape(1, _N, _D)
